# baseline (speedup 1.0000x reference)
import functools

import jax
import jax.numpy as jnp
from jax import lax
from jax.experimental import pallas as pl
from jax.experimental.pallas import tpu as pltpu
from jax.experimental.pallas import tpu_sc as plsc

L = 16
NC = 2
NS = 16
NW = NC * NS
BATCH = 16
PAIRS = 2048
W = 384
IMG = W * W
PPW = BATCH * PAIRS // NW
CH = PPW // L
GCH = 128
NG = PPW // GCH

_mesh = plsc.VectorSubcoreMesh(core_axis_name="c", subcore_axis_name="s")


def _loss_partials_body(img_hbm, ord_hbm, out_hbm,
                        xa_v, ya_v, xb_v, yb_v, r_v,
                        ia_v, ib_v, a_v, b_v, part_v, sem):
    c = lax.axis_index("c")
    s = lax.axis_index("s")
    w = s * NC + c
    n = w // NC
    h = w % NC
    img_off = n * IMG

    psl = pl.ds(h * PPW, PPW)
    ord_copies = [
        pltpu.async_copy(ord_hbm.at[k, n, psl], v, sem)
        for k, v in enumerate((xa_v, ya_v, xb_v, yb_v, r_v))
    ]
    for cp in ord_copies:
        cp.wait()

    def tiled_off(x, y):
        return (
            (x >> 3) * 3072 + (y >> 7) * 1024
            + (x & 7) * 128 + (y & 127) + img_off
        )

    def idx_body(j, carry):
        sl = pl.ds(j * L, L)
        ia_v[sl] = tiled_off(xa_v[sl], ya_v[sl])
        ib_v[sl] = tiled_off(xb_v[sl], yb_v[sl])
        return carry

    lax.fori_loop(0, CH, idx_body, 0, unroll=True)

    copies = []
    for g in range(NG):
        gs = pl.ds(g * GCH, GCH)
        copies.append(pltpu.async_copy(img_hbm.at[ia_v.at[gs]], a_v.at[gs], sem))
        copies.append(pltpu.async_copy(img_hbm.at[ib_v.at[gs]], b_v.at[gs], sem))

    def loss_body(j, acc):
        sl = pl.ds(j * L, L)
        a = a_v[sl]
        b = b_v[sl]
        r = r_v[sl]
        d = a - b
        t = jnp.exp(-jnp.abs(d))
        u = t / (t + 2.0)
        u2 = u * u
        poly = 1.0 + u2 * (1.0 / 3.0 + u2 * (1.0 / 5.0 + u2 * (1.0 / 7.0 + u2 * (1.0 / 9.0))))
        l1p = 2.0 * u * poly
        relu = jnp.maximum(d, 0.0)
        sp_pos = relu + l1p
        sp_neg = relu - d + l1p
        loss = jnp.where(r == 0, d * d, jnp.where(r == 1, sp_pos, sp_neg))
        return acc + loss

    acc = jnp.zeros((L,), jnp.float32)
    for g in range(NG):
        copies[2 * g].wait()
        copies[2 * g + 1].wait()
        acc = lax.fori_loop(g * (GCH // L), (g + 1) * (GCH // L), loss_body,
                            acc, unroll=False)
    part_v[...] = acc
    pltpu.sync_copy(part_v, out_hbm.at[w])


def _tc_mean_body(part_ref, out_ref):
    out_ref[...] = jnp.sum(part_ref[...], keepdims=True) * (1.0 / (BATCH * PAIRS))


_partials_call = pl.kernel(
    _loss_partials_body,
    out_type=jax.ShapeDtypeStruct((NW, L), jnp.float32),
    mesh=_mesh,
    compiler_params=pltpu.CompilerParams(needs_layout_passes=False),
    scratch_types=[
        pltpu.VMEM((PPW,), jnp.int32),
        pltpu.VMEM((PPW,), jnp.int32),
        pltpu.VMEM((PPW,), jnp.int32),
        pltpu.VMEM((PPW,), jnp.int32),
        pltpu.VMEM((PPW,), jnp.int32),
        pltpu.VMEM((PPW,), jnp.int32),
        pltpu.VMEM((PPW,), jnp.int32),
        pltpu.VMEM((PPW,), jnp.float32),
        pltpu.VMEM((PPW,), jnp.float32),
        pltpu.VMEM((L,), jnp.float32),
        pltpu.SemaphoreType.DMA,
    ],
)

_tc_mean_call = pl.pallas_call(
    _tc_mean_body,
    out_shape=jax.ShapeDtypeStruct((1, 1), jnp.float32),
)


@jax.jit
def kernel(output, ordinal):
    img = (
        output.reshape(BATCH, W // 8, 8, W // 128, 128)
        .transpose(0, 1, 3, 2, 4)
        .reshape(BATCH * IMG)
    )
    planes = jnp.transpose(ordinal.astype(jnp.int32), (2, 0, 1))
    partials = _partials_call(img, planes)
    return _tc_mean_call(partials)[0, 0]


# speedup vs baseline: 1.0500x; 1.0500x over previous
import functools

import jax
import jax.numpy as jnp
from jax import lax
from jax.experimental import pallas as pl
from jax.experimental.pallas import tpu as pltpu
from jax.experimental.pallas import tpu_sc as plsc

L = 16
NC = 2
NS = 16
NW = NC * NS
BATCH = 16
PAIRS = 2048
W = 384
IMG = W * W
PPW = BATCH * PAIRS // NW
CH = PPW // L
GCH = 128
NG = PPW // GCH

_mesh = plsc.VectorSubcoreMesh(core_axis_name="c", subcore_axis_name="s")


def _loss_partials_body(img_hbm, ord_hbm, out_hbm,
                        xa_v, ya_v, xb_v, yb_v, r_v,
                        ia_v, ib_v, a_v, b_v, part_v, sem):
    c = lax.axis_index("c")
    s = lax.axis_index("s")
    w = s * NC + c
    n = w // NC
    h = w % NC
    img_off = n * IMG

    psl = pl.ds(h * PPW, PPW)
    ord_copies = [
        pltpu.async_copy(ord_hbm.at[k, n, psl], v, sem)
        for k, v in enumerate((xa_v, ya_v, xb_v, yb_v, r_v))
    ]
    for cp in ord_copies:
        cp.wait()

    def tiled_off(x, y):
        return (
            (x >> 3) * 3072 + (y >> 7) * 1024
            + (x & 7) * 128 + (y & 127) + img_off
        )

    def idx_body(j, carry):
        sl = pl.ds(j * L, L)
        ia_v[sl] = tiled_off(xa_v[sl], ya_v[sl])
        ib_v[sl] = tiled_off(xb_v[sl], yb_v[sl])
        return carry

    lax.fori_loop(0, CH, idx_body, 0, unroll=True)

    copies = []
    for g in range(NG):
        gs = pl.ds(g * GCH, GCH)
        copies.append(pltpu.async_copy(img_hbm.at[ia_v.at[gs]], a_v.at[gs], sem))
        copies.append(pltpu.async_copy(img_hbm.at[ib_v.at[gs]], b_v.at[gs], sem))
    for cp in copies:
        cp.wait()

    def loss_body(j, acc):
        sl = pl.ds(j * L, L)
        a = a_v[sl]
        b = b_v[sl]
        r = r_v[sl]
        d = a - b
        t = jnp.exp(-jnp.abs(d))
        u = t / (t + 2.0)
        u2 = u * u
        poly = 1.0 + u2 * (1.0 / 3.0 + u2 * (1.0 / 5.0 + u2 * (1.0 / 7.0 + u2 * (1.0 / 9.0))))
        l1p = 2.0 * u * poly
        relu = jnp.maximum(d, 0.0)
        sp_pos = relu + l1p
        sp_neg = relu - d + l1p
        loss = jnp.where(r == 0, d * d, jnp.where(r == 1, sp_pos, sp_neg))
        return acc + loss

    acc = lax.fori_loop(0, CH, loss_body, jnp.zeros((L,), jnp.float32),
                        unroll=False)
    part_v[...] = acc
    pltpu.sync_copy(part_v, out_hbm.at[w])


def _tc_mean_body(part_ref, out_ref):
    out_ref[...] = jnp.sum(part_ref[...], keepdims=True) * (1.0 / (BATCH * PAIRS))


_partials_call = pl.kernel(
    _loss_partials_body,
    out_type=jax.ShapeDtypeStruct((NW, L), jnp.float32),
    mesh=_mesh,
    compiler_params=pltpu.CompilerParams(needs_layout_passes=False),
    scratch_types=[
        pltpu.VMEM((PPW,), jnp.int32),
        pltpu.VMEM((PPW,), jnp.int32),
        pltpu.VMEM((PPW,), jnp.int32),
        pltpu.VMEM((PPW,), jnp.int32),
        pltpu.VMEM((PPW,), jnp.int32),
        pltpu.VMEM((PPW,), jnp.int32),
        pltpu.VMEM((PPW,), jnp.int32),
        pltpu.VMEM((PPW,), jnp.float32),
        pltpu.VMEM((PPW,), jnp.float32),
        pltpu.VMEM((L,), jnp.float32),
        pltpu.SemaphoreType.DMA,
    ],
)

_tc_mean_call = pl.pallas_call(
    _tc_mean_body,
    out_shape=jax.ShapeDtypeStruct((1, 1), jnp.float32),
)


@jax.jit
def kernel(output, ordinal):
    img = (
        output.reshape(BATCH, W // 8, 8, W // 128, 128)
        .transpose(0, 1, 3, 2, 4)
        .reshape(BATCH * IMG)
    )
    planes = jnp.transpose(ordinal.astype(jnp.int32), (2, 0, 1))
    partials = _partials_call(img, planes)
    return _tc_mean_call(partials)[0, 0]
